# Initial kernel scaffold; baseline (speedup 1.0000x reference)
import jax
import jax.numpy as jnp
from jax.experimental import pallas as pl


def kernel(x, W_qkv, W_out, Wq_idx, Wk_idx):
    raise NotImplementedError("write your pallas kernel here")


# speedup vs baseline: 127.4405x; 127.4405x over previous
import functools

import jax
import jax.numpy as jnp
import numpy as np
from jax import lax
from jax.experimental import pallas as pl
from jax.experimental.pallas import tpu as pltpu

H = 32
RANK = 128
TOPK = 64

_INT_MIN = np.int32(np.uint32(0x80000000))
_INT_MAXP = np.int32(np.uint32(0x7FFFFFFF))


def _mm_body(a_ref, b_ref, o_ref, acc_ref, *, nk):
    k = pl.program_id(2)

    @pl.when(k == 0)
    def _():
        acc_ref[...] = jnp.zeros_like(acc_ref)

    acc_ref[...] += jnp.dot(a_ref[...], b_ref[...],
                            preferred_element_type=jnp.float32)

    @pl.when(k == nk - 1)
    def _():
        o_ref[...] = acc_ref[...]


def _matmul(a, b, bm=256, bn=512, bk=512):
    m, kk = a.shape
    _, n = b.shape
    bm, bn, bk = min(bm, m), min(bn, n), min(bk, kk)
    nk = kk // bk
    return pl.pallas_call(
        functools.partial(_mm_body, nk=nk),
        grid=(m // bm, n // bn, nk),
        in_specs=[
            pl.BlockSpec((bm, bk), lambda i, j, k: (i, k)),
            pl.BlockSpec((bk, bn), lambda i, j, k: (k, j)),
        ],
        out_specs=pl.BlockSpec((bm, bn), lambda i, j, k: (i, j)),
        out_shape=jax.ShapeDtypeStruct((m, n), jnp.float32),
        scratch_shapes=[pltpu.VMEM((bm, bn), jnp.float32)],
    )(a, b)


def _indexer_body(q_ref, k_ref, wq_ref, wk_ref, qi_ref, ki_ref, *, hd):
    qsum = q_ref[:, 0:hd]
    ksum = k_ref[:, 0:hd]
    for h in range(1, H):
        qsum = qsum + q_ref[:, h * hd:(h + 1) * hd]
        ksum = ksum + k_ref[:, h * hd:(h + 1) * hd]
    qm = qsum * (1.0 / H)
    km = ksum * (1.0 / H)
    qi_ref[...] = jnp.dot(qm, wq_ref[...], preferred_element_type=jnp.float32)
    ki_ref[...] = jnp.dot(km, wk_ref[...], preferred_element_type=jnp.float32)


def _indexer(qkv, wq, wk, s, d, hd, bq=256):
    grid = (s // bq,)
    return pl.pallas_call(
        functools.partial(_indexer_body, hd=hd),
        grid=grid,
        in_specs=[
            pl.BlockSpec((bq, d), lambda i: (i, 0)),
            pl.BlockSpec((bq, d), lambda i: (i, 1)),
            pl.BlockSpec((hd, RANK), lambda i: (0, 0)),
            pl.BlockSpec((hd, RANK), lambda i: (0, 0)),
        ],
        out_specs=[
            pl.BlockSpec((bq, RANK), lambda i: (i, 0)),
            pl.BlockSpec((bq, RANK), lambda i: (i, 0)),
        ],
        out_shape=[
            jax.ShapeDtypeStruct((s, RANK), jnp.float32),
            jax.ShapeDtypeStruct((s, RANK), jnp.float32),
        ],
    )(qkv, qkv, wq, wk)


def _mask_body(qi_ref, ki_ref, mask_ref):
    scores = lax.dot_general(
        qi_ref[...], ki_ref[...], (((1,), (1,)), ((), ())),
        preferred_element_type=jnp.float32) * (1.0 / np.sqrt(RANK))
    bits = lax.bitcast_convert_type(scores, jnp.int32)
    skey = jnp.where(bits >= 0, bits, bits ^ _INT_MAXP)
    t = jnp.zeros(skey.shape[:1] + (1,), jnp.int32)
    for bit in range(31, -1, -1):
        cand = t | np.int32(np.uint32(1 << bit))
        scand = cand ^ _INT_MIN
        cnt = jnp.sum((skey >= scand).astype(jnp.int32), axis=1,
                      keepdims=True)
        t = jnp.where(cnt >= TOPK, cand, t)
    mask_ref[...] = (skey >= (t ^ _INT_MIN)).astype(jnp.int8)


def _topk_mask(qi, ki, s, bq=256):
    return pl.pallas_call(
        _mask_body,
        grid=(s // bq,),
        in_specs=[
            pl.BlockSpec((bq, RANK), lambda i: (i, 0)),
            pl.BlockSpec((s, RANK), lambda i: (0, 0)),
        ],
        out_specs=pl.BlockSpec((bq, s), lambda i: (i, 0)),
        out_shape=jax.ShapeDtypeStruct((s, s), jnp.int8),
    )(qi, ki)


def _attn_body(q_ref, k_ref, v_ref, m_ref, o_ref, *, scale, hd, hpb):
    msk = m_ref[...] != 0
    for h in range(hpb):
        q = q_ref[:, h * hd:(h + 1) * hd]
        k = k_ref[:, h * hd:(h + 1) * hd]
        s = lax.dot_general(q, k, (((1,), (1,)), ((), ())),
                            preferred_element_type=jnp.float32) * scale
        s = jnp.where(msk, s, -1e30)
        mx = jnp.max(s, axis=1, keepdims=True)
        e = jnp.exp(s - mx)
        p = e / jnp.sum(e, axis=1, keepdims=True)
        o_ref[:, h * hd:(h + 1) * hd] = jnp.dot(
            p, v_ref[:, h * hd:(h + 1) * hd],
            preferred_element_type=jnp.float32)


def _attention(qkv, mask, s, hd, bq=256, hpb=2):
    scale = 1.0 / np.sqrt(hd)
    w = hpb * hd
    npair = H // hpb
    return pl.pallas_call(
        functools.partial(_attn_body, scale=scale, hd=hd, hpb=hpb),
        grid=(npair, s // bq),
        in_specs=[
            pl.BlockSpec((bq, w), lambda h, i: (i, h)),
            pl.BlockSpec((s, w), lambda h, i: (0, npair + h)),
            pl.BlockSpec((s, w), lambda h, i: (0, 2 * npair + h)),
            pl.BlockSpec((bq, s), lambda h, i: (i, 0)),
        ],
        out_specs=pl.BlockSpec((bq, w), lambda h, i: (i, h)),
        out_shape=jax.ShapeDtypeStruct((s, H * hd), jnp.float32),
    )(qkv, qkv, qkv, mask)


def kernel(x, W_qkv, W_out, Wq_idx, Wk_idx):
    b, s, d = x.shape
    hd = d // H
    x2 = x.reshape(s, d)
    qkv = _matmul(x2, W_qkv)
    qi, ki = _indexer(qkv, Wq_idx, Wk_idx, s, d, hd)
    mask = _topk_mask(qi, ki, s)
    attn = _attention(qkv, mask, s, hd)
    out = _matmul(attn, W_out)
    return out.reshape(b, s, d)


# speedup vs baseline: 130.7592x; 1.0260x over previous
import functools

import jax
import jax.numpy as jnp
import numpy as np
from jax import lax
from jax.experimental import pallas as pl
from jax.experimental.pallas import tpu as pltpu

H = 32
RANK = 128
TOPK = 64

_INT_MIN = np.int32(np.uint32(0x80000000))
_INT_MAXP = np.int32(np.uint32(0x7FFFFFFF))


def _mm_body(a_ref, b_ref, o_ref, acc_ref, *, nk):
    k = pl.program_id(2)

    @pl.when(k == 0)
    def _():
        acc_ref[...] = jnp.zeros_like(acc_ref)

    acc_ref[...] += jnp.dot(a_ref[...].astype(jnp.bfloat16),
                            b_ref[...].astype(jnp.bfloat16),
                            preferred_element_type=jnp.float32)

    @pl.when(k == nk - 1)
    def _():
        o_ref[...] = acc_ref[...]


def _matmul(a, b, bm=256, bn=512, bk=512):
    m, kk = a.shape
    _, n = b.shape
    bm, bn, bk = min(bm, m), min(bn, n), min(bk, kk)
    nk = kk // bk
    return pl.pallas_call(
        functools.partial(_mm_body, nk=nk),
        grid=(m // bm, n // bn, nk),
        in_specs=[
            pl.BlockSpec((bm, bk), lambda i, j, k: (i, k)),
            pl.BlockSpec((bk, bn), lambda i, j, k: (k, j)),
        ],
        out_specs=pl.BlockSpec((bm, bn), lambda i, j, k: (i, j)),
        out_shape=jax.ShapeDtypeStruct((m, n), jnp.float32),
        scratch_shapes=[pltpu.VMEM((bm, bn), jnp.float32)],
    )(a, b)


def _indexer_body(q_ref, k_ref, wq_ref, wk_ref, qi_ref, ki_ref, *, hd):
    qsum = q_ref[:, 0:hd]
    ksum = k_ref[:, 0:hd]
    for h in range(1, H):
        qsum = qsum + q_ref[:, h * hd:(h + 1) * hd]
        ksum = ksum + k_ref[:, h * hd:(h + 1) * hd]
    qm = qsum * (1.0 / H)
    km = ksum * (1.0 / H)
    qi_ref[...] = jnp.dot(qm, wq_ref[...], preferred_element_type=jnp.float32)
    ki_ref[...] = jnp.dot(km, wk_ref[...], preferred_element_type=jnp.float32)


def _indexer(qkv, wq, wk, s, d, hd, bq=256):
    grid = (s // bq,)
    return pl.pallas_call(
        functools.partial(_indexer_body, hd=hd),
        grid=grid,
        in_specs=[
            pl.BlockSpec((bq, d), lambda i: (i, 0)),
            pl.BlockSpec((bq, d), lambda i: (i, 1)),
            pl.BlockSpec((hd, RANK), lambda i: (0, 0)),
            pl.BlockSpec((hd, RANK), lambda i: (0, 0)),
        ],
        out_specs=[
            pl.BlockSpec((bq, RANK), lambda i: (i, 0)),
            pl.BlockSpec((bq, RANK), lambda i: (i, 0)),
        ],
        out_shape=[
            jax.ShapeDtypeStruct((s, RANK), jnp.float32),
            jax.ShapeDtypeStruct((s, RANK), jnp.float32),
        ],
    )(qkv, qkv, wq, wk)


def _mask_body(qi_ref, ki_ref, mask_ref):
    scores = lax.dot_general(
        qi_ref[...], ki_ref[...], (((1,), (1,)), ((), ())),
        preferred_element_type=jnp.float32) * (1.0 / np.sqrt(RANK))
    bits = lax.bitcast_convert_type(scores, jnp.int32)
    skey = jnp.where(bits >= 0, bits, bits ^ _INT_MAXP)
    t = jnp.zeros(skey.shape[:1] + (1,), jnp.int32)
    for bit in range(31, -1, -1):
        cand = t | np.int32(np.uint32(1 << bit))
        scand = cand ^ _INT_MIN
        cnt = jnp.sum((skey >= scand).astype(jnp.int32), axis=1,
                      keepdims=True)
        t = jnp.where(cnt >= TOPK, cand, t)
    mask_ref[...] = (skey >= (t ^ _INT_MIN)).astype(jnp.int8)


def _topk_mask(qi, ki, s, bq=256):
    return pl.pallas_call(
        _mask_body,
        grid=(s // bq,),
        in_specs=[
            pl.BlockSpec((bq, RANK), lambda i: (i, 0)),
            pl.BlockSpec((s, RANK), lambda i: (0, 0)),
        ],
        out_specs=pl.BlockSpec((bq, s), lambda i: (i, 0)),
        out_shape=jax.ShapeDtypeStruct((s, s), jnp.int8),
    )(qi, ki)


def _attn_body(q_ref, k_ref, v_ref, m_ref, o_ref, *, scale, hd, hpb):
    msk = m_ref[...] != 0
    for h in range(hpb):
        q = q_ref[:, h * hd:(h + 1) * hd].astype(jnp.bfloat16)
        k = k_ref[:, h * hd:(h + 1) * hd].astype(jnp.bfloat16)
        s = lax.dot_general(q, k, (((1,), (1,)), ((), ())),
                            preferred_element_type=jnp.float32) * scale
        s = jnp.where(msk, s, -1e30)
        mx = jnp.max(s, axis=1, keepdims=True)
        e = jnp.exp(s - mx)
        p = (e / jnp.sum(e, axis=1, keepdims=True)).astype(jnp.bfloat16)
        o_ref[:, h * hd:(h + 1) * hd] = jnp.dot(
            p, v_ref[:, h * hd:(h + 1) * hd].astype(jnp.bfloat16),
            preferred_element_type=jnp.float32)


def _attention(qkv, mask, s, hd, bq=256, hpb=2):
    scale = 1.0 / np.sqrt(hd)
    w = hpb * hd
    npair = H // hpb
    return pl.pallas_call(
        functools.partial(_attn_body, scale=scale, hd=hd, hpb=hpb),
        grid=(npair, s // bq),
        in_specs=[
            pl.BlockSpec((bq, w), lambda h, i: (i, h)),
            pl.BlockSpec((s, w), lambda h, i: (0, npair + h)),
            pl.BlockSpec((s, w), lambda h, i: (0, 2 * npair + h)),
            pl.BlockSpec((bq, s), lambda h, i: (i, 0)),
        ],
        out_specs=pl.BlockSpec((bq, w), lambda h, i: (i, h)),
        out_shape=jax.ShapeDtypeStruct((s, H * hd), jnp.float32),
    )(qkv, qkv, qkv, mask)


def kernel(x, W_qkv, W_out, Wq_idx, Wk_idx):
    b, s, d = x.shape
    hd = d // H
    x2 = x.reshape(s, d)
    qkv = _matmul(x2, W_qkv)
    qi, ki = _indexer(qkv, Wq_idx, Wk_idx, s, d, hd)
    mask = _topk_mask(qi, ki, s)
    attn = _attention(qkv, mask, s, hd)
    out = _matmul(attn, W_out)
    return out.reshape(b, s, d)


# speedup vs baseline: 259.0029x; 1.9808x over previous
import functools

import jax
import jax.numpy as jnp
import numpy as np
from jax import lax
from jax.experimental import pallas as pl
from jax.experimental.pallas import tpu as pltpu

H = 32
RANK = 128
TOPK = 64

_INT_MIN = np.int32(np.uint32(0x80000000))
_INT_MAXP = np.int32(np.uint32(0x7FFFFFFF))


def _mm_body(a_ref, b_ref, o_ref, *, cdt):
    o_ref[...] = jnp.dot(a_ref[...].astype(cdt), b_ref[...].astype(cdt),
                         preferred_element_type=jnp.float32)


def _matmul(a, b, bn=512, cdt=jnp.float32):
    m, kk = a.shape
    _, n = b.shape
    return pl.pallas_call(
        functools.partial(_mm_body, cdt=cdt),
        grid=(n // bn,),
        in_specs=[
            pl.BlockSpec((m, kk), lambda j: (0, 0)),
            pl.BlockSpec((kk, bn), lambda j: (0, j)),
        ],
        out_specs=pl.BlockSpec((m, bn), lambda j: (0, j)),
        out_shape=jax.ShapeDtypeStruct((m, n), jnp.float32),
    )(a, b)


def _indexer_body(q_ref, k_ref, wq_ref, wk_ref, qi_ref, ki_ref, *, hd):
    qsum = q_ref[:, 0:hd]
    ksum = k_ref[:, 0:hd]
    for h in range(1, H):
        qsum = qsum + q_ref[:, h * hd:(h + 1) * hd]
        ksum = ksum + k_ref[:, h * hd:(h + 1) * hd]
    qm = qsum * (1.0 / H)
    km = ksum * (1.0 / H)
    qi_ref[...] = jnp.dot(qm, wq_ref[...], preferred_element_type=jnp.float32)
    ki_ref[...] = jnp.dot(km, wk_ref[...], preferred_element_type=jnp.float32)


def _indexer(qkv, wq, wk, s, d, hd, bq=256):
    grid = (s // bq,)
    return pl.pallas_call(
        functools.partial(_indexer_body, hd=hd),
        grid=grid,
        in_specs=[
            pl.BlockSpec((bq, d), lambda i: (i, 0)),
            pl.BlockSpec((bq, d), lambda i: (i, 1)),
            pl.BlockSpec((hd, RANK), lambda i: (0, 0)),
            pl.BlockSpec((hd, RANK), lambda i: (0, 0)),
        ],
        out_specs=[
            pl.BlockSpec((bq, RANK), lambda i: (i, 0)),
            pl.BlockSpec((bq, RANK), lambda i: (i, 0)),
        ],
        out_shape=[
            jax.ShapeDtypeStruct((s, RANK), jnp.float32),
            jax.ShapeDtypeStruct((s, RANK), jnp.float32),
        ],
    )(qkv, qkv, wq, wk)


def _mask_body(qi_ref, ki_ref, mask_ref):
    scores = lax.dot_general(
        qi_ref[...], ki_ref[...], (((1,), (1,)), ((), ())),
        preferred_element_type=jnp.float32) * (1.0 / np.sqrt(RANK))
    bits = lax.bitcast_convert_type(scores, jnp.int32)
    skey = jnp.where(bits >= 0, bits, bits ^ _INT_MAXP)
    t = jnp.zeros(skey.shape[:1] + (1,), jnp.int32)
    for bit in range(31, -1, -1):
        cand = t | np.int32(np.uint32(1 << bit))
        scand = cand ^ _INT_MIN
        cnt = jnp.sum((skey >= scand).astype(jnp.int32), axis=1,
                      keepdims=True)
        t = jnp.where(cnt >= TOPK, cand, t)
    mask_ref[...] = (skey >= (t ^ _INT_MIN)).astype(jnp.int8)


def _topk_mask(qi, ki, s, bq=256):
    return pl.pallas_call(
        _mask_body,
        grid=(s // bq,),
        in_specs=[
            pl.BlockSpec((bq, RANK), lambda i: (i, 0)),
            pl.BlockSpec((s, RANK), lambda i: (0, 0)),
        ],
        out_specs=pl.BlockSpec((bq, s), lambda i: (i, 0)),
        out_shape=jax.ShapeDtypeStruct((s, s), jnp.int8),
    )(qi, ki)


def _attn_body(q_ref, k_ref, v_ref, m_ref, o_ref, *, scale, hd, hpb):
    msk = m_ref[...] != 0
    for h in range(hpb):
        q = q_ref[:, h * hd:(h + 1) * hd].astype(jnp.bfloat16)
        k = k_ref[:, h * hd:(h + 1) * hd].astype(jnp.bfloat16)
        s = lax.dot_general(q, k, (((1,), (1,)), ((), ())),
                            preferred_element_type=jnp.float32) * scale
        s = jnp.where(msk, s, -1e30)
        mx = jnp.max(s, axis=1, keepdims=True)
        e = jnp.exp(s - mx)
        inv = 1.0 / jnp.sum(e, axis=1, keepdims=True)
        p = (e * inv).astype(jnp.bfloat16)
        o_ref[:, h * hd:(h + 1) * hd] = jnp.dot(
            p, v_ref[:, h * hd:(h + 1) * hd].astype(jnp.bfloat16),
            preferred_element_type=jnp.float32)


def _attention(qkv, mask, s, hd, bq=512, hpb=4):
    scale = 1.0 / np.sqrt(hd)
    bq = min(bq, s)
    w = hpb * hd
    npair = H // hpb
    return pl.pallas_call(
        functools.partial(_attn_body, scale=scale, hd=hd, hpb=hpb),
        grid=(npair, s // bq),
        in_specs=[
            pl.BlockSpec((bq, w), lambda h, i: (i, h)),
            pl.BlockSpec((s, w), lambda h, i: (0, npair + h)),
            pl.BlockSpec((s, w), lambda h, i: (0, 2 * npair + h)),
            pl.BlockSpec((bq, s), lambda h, i: (i, 0)),
        ],
        out_specs=pl.BlockSpec((bq, w), lambda h, i: (i, h)),
        out_shape=jax.ShapeDtypeStruct((s, H * hd), jnp.float32),
    )(qkv, qkv, qkv, mask)


def kernel(x, W_qkv, W_out, Wq_idx, Wk_idx):
    b, s, d = x.shape
    hd = d // H
    x2 = x.reshape(s, d)
    qkv = _matmul(x2, W_qkv)
    qi, ki = _indexer(qkv, Wq_idx, Wk_idx, s, d, hd)
    mask = _topk_mask(qi, ki, s)
    attn = _attention(qkv, mask, s, hd)
    out = _matmul(attn, W_out)
    return out.reshape(b, s, d)


# speedup vs baseline: 293.4952x; 1.1332x over previous
import functools

import jax
import jax.numpy as jnp
import numpy as np
from jax import lax
from jax.experimental import pallas as pl
from jax.experimental.pallas import tpu as pltpu

H = 32
RANK = 128
TOPK = 64

_INT_MIN = np.int32(np.uint32(0x80000000))
_INT_MAXP = np.int32(np.uint32(0x7FFFFFFF))


def _mm_body(a_ref, b_ref, o_ref, *, cdt):
    o_ref[...] = jnp.dot(a_ref[...].astype(cdt), b_ref[...].astype(cdt),
                         preferred_element_type=jnp.float32)


def _matmul(a, b, bn=512, cdt=jnp.float32):
    m, kk = a.shape
    _, n = b.shape
    return pl.pallas_call(
        functools.partial(_mm_body, cdt=cdt),
        grid=(n // bn,),
        in_specs=[
            pl.BlockSpec((m, kk), lambda j: (0, 0)),
            pl.BlockSpec((kk, bn), lambda j: (0, j)),
        ],
        out_specs=pl.BlockSpec((m, bn), lambda j: (0, j)),
        out_shape=jax.ShapeDtypeStruct((m, n), jnp.float32),
    )(a, b)


def _indexer_body(q_ref, k_ref, wq_ref, wk_ref, qi_ref, ki_ref, *, hd):
    qsum = q_ref[:, 0:hd]
    ksum = k_ref[:, 0:hd]
    for h in range(1, H):
        qsum = qsum + q_ref[:, h * hd:(h + 1) * hd]
        ksum = ksum + k_ref[:, h * hd:(h + 1) * hd]
    qm = qsum * (1.0 / H)
    km = ksum * (1.0 / H)
    qi_ref[...] = jnp.dot(qm, wq_ref[...], preferred_element_type=jnp.float32)
    ki_ref[...] = jnp.dot(km, wk_ref[...], preferred_element_type=jnp.float32)


def _indexer(qkv, wq, wk, s, d, hd, bq=256):
    grid = (s // bq,)
    return pl.pallas_call(
        functools.partial(_indexer_body, hd=hd),
        grid=grid,
        in_specs=[
            pl.BlockSpec((bq, d), lambda i: (i, 0)),
            pl.BlockSpec((bq, d), lambda i: (i, 1)),
            pl.BlockSpec((hd, RANK), lambda i: (0, 0)),
            pl.BlockSpec((hd, RANK), lambda i: (0, 0)),
        ],
        out_specs=[
            pl.BlockSpec((bq, RANK), lambda i: (i, 0)),
            pl.BlockSpec((bq, RANK), lambda i: (i, 0)),
        ],
        out_shape=[
            jax.ShapeDtypeStruct((s, RANK), jnp.float32),
            jax.ShapeDtypeStruct((s, RANK), jnp.float32),
        ],
    )(qkv, qkv, wq, wk)


def _mask_body(qi_ref, ki_ref, mask_ref):
    scores = lax.dot_general(
        qi_ref[...], ki_ref[...], (((1,), (1,)), ((), ())),
        preferred_element_type=jnp.float32) * (1.0 / np.sqrt(RANK))
    bits = lax.bitcast_convert_type(scores, jnp.int32)
    skey = jnp.where(bits >= 0, bits, bits ^ _INT_MAXP)
    t = jnp.zeros(skey.shape[:1] + (1,), jnp.int32)
    for bit in range(31, -1, -1):
        cand = t | np.int32(np.uint32(1 << bit))
        scand = cand ^ _INT_MIN
        cnt = jnp.sum((skey >= scand).astype(jnp.int32), axis=1,
                      keepdims=True)
        t = jnp.where(cnt >= TOPK, cand, t)
    mask_ref[...] = (skey >= (t ^ _INT_MIN)).astype(jnp.int8)


def _topk_mask(qi, ki, s, bq=256):
    return pl.pallas_call(
        _mask_body,
        grid=(s // bq,),
        in_specs=[
            pl.BlockSpec((bq, RANK), lambda i: (i, 0)),
            pl.BlockSpec((s, RANK), lambda i: (0, 0)),
        ],
        out_specs=pl.BlockSpec((bq, s), lambda i: (i, 0)),
        out_shape=jax.ShapeDtypeStruct((s, s), jnp.int8),
    )(qi, ki)


def _attn_body(q_ref, k_ref, v_ref, m_ref, o_ref, *, scale, hd, hpb):
    msk = m_ref[...] != 0
    for h in range(hpb):
        q = (q_ref[:, h * hd:(h + 1) * hd] * scale).astype(jnp.bfloat16)
        k = k_ref[:, h * hd:(h + 1) * hd].astype(jnp.bfloat16)
        s = lax.dot_general(q, k, (((1,), (1,)), ((), ())),
                            preferred_element_type=jnp.float32)
        s = jnp.where(msk, s, -1e30)
        mx = jnp.max(s, axis=1, keepdims=True)
        e32 = jnp.exp(s - mx)
        e = e32.astype(jnp.bfloat16)
        inv = 1.0 / jnp.sum(e32, axis=1, keepdims=True)
        o = jnp.dot(e, v_ref[:, h * hd:(h + 1) * hd].astype(jnp.bfloat16),
                    preferred_element_type=jnp.float32)
        o_ref[:, h * hd:(h + 1) * hd] = o * inv


def _attention(qkv, mask, s, hd, bq=512, hpb=4):
    scale = 1.0 / np.sqrt(hd)
    bq = min(bq, s)
    w = hpb * hd
    npair = H // hpb
    return pl.pallas_call(
        functools.partial(_attn_body, scale=scale, hd=hd, hpb=hpb),
        grid=(npair, s // bq),
        in_specs=[
            pl.BlockSpec((bq, w), lambda h, i: (i, h)),
            pl.BlockSpec((s, w), lambda h, i: (0, npair + h)),
            pl.BlockSpec((s, w), lambda h, i: (0, 2 * npair + h)),
            pl.BlockSpec((bq, s), lambda h, i: (i, 0)),
        ],
        out_specs=pl.BlockSpec((bq, w), lambda h, i: (i, h)),
        out_shape=jax.ShapeDtypeStruct((s, H * hd), jnp.float32),
    )(qkv, qkv, qkv, mask)


def kernel(x, W_qkv, W_out, Wq_idx, Wk_idx):
    b, s, d = x.shape
    hd = d // H
    x2 = x.reshape(s, d)
    qkv = _matmul(x2, W_qkv)
    qi, ki = _indexer(qkv, Wq_idx, Wk_idx, s, d, hd)
    mask = _topk_mask(qi, ki, s)
    attn = _attention(qkv, mask, s, hd)
    out = _matmul(attn, W_out)
    return out.reshape(b, s, d)


# speedup vs baseline: 309.4454x; 1.0543x over previous
import functools

import jax
import jax.numpy as jnp
import numpy as np
from jax import lax
from jax.experimental import pallas as pl
from jax.experimental.pallas import tpu as pltpu

H = 32
RANK = 128
TOPK = 64

_INT_MIN = np.int32(np.uint32(0x80000000))
_INT_MAXP = np.int32(np.uint32(0x7FFFFFFF))


def _mm_body(a_ref, b_ref, o_ref, *, cdt):
    o_ref[...] = jnp.dot(a_ref[...].astype(cdt), b_ref[...].astype(cdt),
                         preferred_element_type=jnp.float32)


def _matmul(a, b, bn=512, cdt=jnp.float32):
    m, kk = a.shape
    _, n = b.shape
    return pl.pallas_call(
        functools.partial(_mm_body, cdt=cdt),
        grid=(n // bn,),
        in_specs=[
            pl.BlockSpec((m, kk), lambda j: (0, 0)),
            pl.BlockSpec((kk, bn), lambda j: (0, j)),
        ],
        out_specs=pl.BlockSpec((m, bn), lambda j: (0, j)),
        out_shape=jax.ShapeDtypeStruct((m, n), jnp.float32),
    )(a, b)


def _indexer_body(q_ref, k_ref, wq_ref, wk_ref, qi_ref, ki_ref, *, hd):
    qsum = q_ref[:, 0:hd]
    ksum = k_ref[:, 0:hd]
    for h in range(1, H):
        qsum = qsum + q_ref[:, h * hd:(h + 1) * hd]
        ksum = ksum + k_ref[:, h * hd:(h + 1) * hd]
    qm = qsum * (1.0 / H)
    km = ksum * (1.0 / H)
    qi_ref[...] = jnp.dot(qm, wq_ref[...], preferred_element_type=jnp.float32)
    ki_ref[...] = jnp.dot(km, wk_ref[...], preferred_element_type=jnp.float32)


def _indexer(qkv, wq, wk, s, d, hd, bq=256):
    grid = (s // bq,)
    return pl.pallas_call(
        functools.partial(_indexer_body, hd=hd),
        grid=grid,
        in_specs=[
            pl.BlockSpec((bq, d), lambda i: (i, 0)),
            pl.BlockSpec((bq, d), lambda i: (i, 1)),
            pl.BlockSpec((hd, RANK), lambda i: (0, 0)),
            pl.BlockSpec((hd, RANK), lambda i: (0, 0)),
        ],
        out_specs=[
            pl.BlockSpec((bq, RANK), lambda i: (i, 0)),
            pl.BlockSpec((bq, RANK), lambda i: (i, 0)),
        ],
        out_shape=[
            jax.ShapeDtypeStruct((s, RANK), jnp.float32),
            jax.ShapeDtypeStruct((s, RANK), jnp.float32),
        ],
    )(qkv, qkv, wq, wk)


def _mask_body(qi_ref, ki_ref, mask_ref):
    scores = lax.dot_general(
        qi_ref[...], ki_ref[...], (((1,), (1,)), ((), ())),
        preferred_element_type=jnp.float32) * (1.0 / np.sqrt(RANK))
    bits = lax.bitcast_convert_type(scores, jnp.int32)
    skey = jnp.where(bits >= 0, bits, bits ^ _INT_MAXP)
    t = jnp.zeros(skey.shape[:1] + (1,), jnp.int32)
    for bit in range(31, -1, -1):
        cand = t | np.int32(np.uint32(1 << bit))
        scand = cand ^ _INT_MIN
        cnt = jnp.sum((skey >= scand).astype(jnp.int32), axis=1,
                      keepdims=True)
        t = jnp.where(cnt >= TOPK, cand, t)
    mask_ref[...] = (skey >= (t ^ _INT_MIN)).astype(jnp.int8)


def _topk_mask(qi, ki, s, bq=256):
    return pl.pallas_call(
        _mask_body,
        grid=(s // bq,),
        in_specs=[
            pl.BlockSpec((bq, RANK), lambda i: (i, 0)),
            pl.BlockSpec((s, RANK), lambda i: (0, 0)),
        ],
        out_specs=pl.BlockSpec((bq, s), lambda i: (i, 0)),
        out_shape=jax.ShapeDtypeStruct((s, s), jnp.int8),
    )(qi, ki)


def _attn_body(q_ref, k_ref, v_ref, m_ref, o_ref, *, scale, hd, hpb):
    msk = m_ref[...] != 0
    for h in range(hpb):
        q = (q_ref[:, h * hd:(h + 1) * hd] * scale).astype(jnp.bfloat16)
        k = k_ref[:, h * hd:(h + 1) * hd].astype(jnp.bfloat16)
        s = lax.dot_general(q, k, (((1,), (1,)), ((), ())),
                            preferred_element_type=jnp.float32)
        s = jnp.where(msk, s, -1e30)
        mx = jnp.max(s, axis=1, keepdims=True)
        e32 = jnp.exp(s - mx)
        e = e32.astype(jnp.bfloat16)
        inv = 1.0 / jnp.sum(e32, axis=1, keepdims=True)
        o = jnp.dot(e, v_ref[:, h * hd:(h + 1) * hd].astype(jnp.bfloat16),
                    preferred_element_type=jnp.float32)
        o_ref[:, h * hd:(h + 1) * hd] = o * inv


def _attention(qkv, mask, s, hd, bq=512, hpb=8):
    scale = 1.0 / np.sqrt(hd)
    bq = min(bq, s)
    w = hpb * hd
    npair = H // hpb
    return pl.pallas_call(
        functools.partial(_attn_body, scale=scale, hd=hd, hpb=hpb),
        grid=(npair, s // bq),
        in_specs=[
            pl.BlockSpec((bq, w), lambda h, i: (i, h)),
            pl.BlockSpec((s, w), lambda h, i: (0, npair + h)),
            pl.BlockSpec((s, w), lambda h, i: (0, 2 * npair + h)),
            pl.BlockSpec((bq, s), lambda h, i: (i, 0)),
        ],
        out_specs=pl.BlockSpec((bq, w), lambda h, i: (i, h)),
        out_shape=jax.ShapeDtypeStruct((s, H * hd), jnp.float32),
    )(qkv, qkv, qkv, mask)


def kernel(x, W_qkv, W_out, Wq_idx, Wk_idx):
    b, s, d = x.shape
    hd = d // H
    x2 = x.reshape(s, d)
    qkv = _matmul(x2, W_qkv)
    qi, ki = _indexer(qkv, Wq_idx, Wk_idx, s, d, hd)
    mask = _topk_mask(qi, ki, s)
    attn = _attention(qkv, mask, s, hd)
    out = _matmul(attn, W_out)
    return out.reshape(b, s, d)


# speedup vs baseline: 317.6639x; 1.0266x over previous
import functools

import jax
import jax.numpy as jnp
import numpy as np
from jax import lax
from jax.experimental import pallas as pl
from jax.experimental.pallas import tpu as pltpu

H = 32
RANK = 128
TOPK = 64

_INT_MIN = np.int32(np.uint32(0x80000000))
_INT_MAXP = np.int32(np.uint32(0x7FFFFFFF))


def _mm_body(a_ref, b_ref, o_ref, *, cdt):
    o_ref[...] = jnp.dot(a_ref[...].astype(cdt), b_ref[...].astype(cdt),
                         preferred_element_type=jnp.float32)


def _matmul(a, b, bn=512, cdt=jnp.float32):
    m, kk = a.shape
    _, n = b.shape
    return pl.pallas_call(
        functools.partial(_mm_body, cdt=cdt),
        grid=(n // bn,),
        in_specs=[
            pl.BlockSpec((m, kk), lambda j: (0, 0)),
            pl.BlockSpec((kk, bn), lambda j: (0, j)),
        ],
        out_specs=pl.BlockSpec((m, bn), lambda j: (0, j)),
        out_shape=jax.ShapeDtypeStruct((m, n), jnp.float32),
    )(a, b)


def _indexer_body(q_ref, k_ref, wq_ref, wk_ref, qi_ref, ki_ref, *, hd):
    qsum = q_ref[:, 0:hd]
    ksum = k_ref[:, 0:hd]
    for h in range(1, H):
        qsum = qsum + q_ref[:, h * hd:(h + 1) * hd]
        ksum = ksum + k_ref[:, h * hd:(h + 1) * hd]
    qm = qsum * (1.0 / H)
    km = ksum * (1.0 / H)
    qi_ref[...] = jnp.dot(qm, wq_ref[...], preferred_element_type=jnp.float32)
    ki_ref[...] = jnp.dot(km, wk_ref[...], preferred_element_type=jnp.float32)


def _indexer(qkv, wq, wk, s, d, hd, bq=256):
    grid = (s // bq,)
    return pl.pallas_call(
        functools.partial(_indexer_body, hd=hd),
        grid=grid,
        in_specs=[
            pl.BlockSpec((bq, d), lambda i: (i, 0)),
            pl.BlockSpec((bq, d), lambda i: (i, 1)),
            pl.BlockSpec((hd, RANK), lambda i: (0, 0)),
            pl.BlockSpec((hd, RANK), lambda i: (0, 0)),
        ],
        out_specs=[
            pl.BlockSpec((bq, RANK), lambda i: (i, 0)),
            pl.BlockSpec((bq, RANK), lambda i: (i, 0)),
        ],
        out_shape=[
            jax.ShapeDtypeStruct((s, RANK), jnp.float32),
            jax.ShapeDtypeStruct((s, RANK), jnp.float32),
        ],
    )(qkv, qkv, wq, wk)


def _mask_body(qi_ref, ki_ref, mask_ref):
    scores = lax.dot_general(
        qi_ref[...], ki_ref[...], (((1,), (1,)), ((), ())),
        preferred_element_type=jnp.float32) * (1.0 / np.sqrt(RANK))
    bits = lax.bitcast_convert_type(scores, jnp.int32)
    skey = jnp.where(bits >= 0, bits, bits ^ _INT_MAXP)
    t = jnp.zeros(skey.shape[:1] + (1,), jnp.int32)
    for bit in range(31, -1, -1):
        cand = t | np.int32(np.uint32(1 << bit))
        scand = cand ^ _INT_MIN
        cnt = jnp.sum((skey >= scand).astype(jnp.int32), axis=1,
                      keepdims=True)
        t = jnp.where(cnt >= TOPK, cand, t)
    mask_ref[...] = (skey >= (t ^ _INT_MIN)).astype(jnp.int8)


def _topk_mask(qi, ki, s, bq=256):
    return pl.pallas_call(
        _mask_body,
        grid=(s // bq,),
        in_specs=[
            pl.BlockSpec((bq, RANK), lambda i: (i, 0)),
            pl.BlockSpec((s, RANK), lambda i: (0, 0)),
        ],
        out_specs=pl.BlockSpec((bq, s), lambda i: (i, 0)),
        out_shape=jax.ShapeDtypeStruct((s, s), jnp.int8),
    )(qi, ki)


def _attn_body(q_ref, k_ref, v_ref, m_ref, o_ref, *, scale, hd, hpb):
    msk = m_ref[...] != 0
    for h in range(hpb):
        q = (q_ref[:, h * hd:(h + 1) * hd] * scale).astype(jnp.bfloat16)
        k = k_ref[:, h * hd:(h + 1) * hd].astype(jnp.bfloat16)
        s = lax.dot_general(q, k, (((1,), (1,)), ((), ())),
                            preferred_element_type=jnp.float32)
        s = jnp.where(msk, s, -1e30)
        mx = jnp.max(s, axis=1, keepdims=True)
        e = jnp.exp(s - mx).astype(jnp.bfloat16)
        inv = 1.0 / jnp.sum(e.astype(jnp.float32), axis=1, keepdims=True)
        o = jnp.dot(e, v_ref[:, h * hd:(h + 1) * hd].astype(jnp.bfloat16),
                    preferred_element_type=jnp.float32)
        o_ref[:, h * hd:(h + 1) * hd] = o * inv


def _attention(qkv, mask, s, hd, bq=512, hpb=8):
    scale = 1.0 / np.sqrt(hd)
    bq = min(bq, s)
    w = hpb * hd
    npair = H // hpb
    return pl.pallas_call(
        functools.partial(_attn_body, scale=scale, hd=hd, hpb=hpb),
        grid=(npair, s // bq),
        in_specs=[
            pl.BlockSpec((bq, w), lambda h, i: (i, h)),
            pl.BlockSpec((s, w), lambda h, i: (0, npair + h)),
            pl.BlockSpec((s, w), lambda h, i: (0, 2 * npair + h)),
            pl.BlockSpec((bq, s), lambda h, i: (i, 0)),
        ],
        out_specs=pl.BlockSpec((bq, w), lambda h, i: (i, h)),
        out_shape=jax.ShapeDtypeStruct((s, H * hd), jnp.float32),
    )(qkv, qkv, qkv, mask)


def kernel(x, W_qkv, W_out, Wq_idx, Wk_idx):
    b, s, d = x.shape
    hd = d // H
    x2 = x.reshape(s, d)
    qkv = _matmul(x2, W_qkv)
    qi, ki = _indexer(qkv, Wq_idx, Wk_idx, s, d, hd)
    mask = _topk_mask(qi, ki, s)
    attn = _attention(qkv, mask, s, hd)
    out = _matmul(attn, W_out)
    return out.reshape(b, s, d)


# speedup vs baseline: 334.7358x; 1.0537x over previous
import functools

import jax
import jax.numpy as jnp
import numpy as np
from jax import lax
from jax.experimental import pallas as pl
from jax.experimental.pallas import tpu as pltpu

H = 32
RANK = 128
TOPK = 64

_INT_MIN = np.int32(np.uint32(0x80000000))
_INT_MAXP = np.int32(np.uint32(0x7FFFFFFF))


def _mm_body(a_ref, b_ref, o_ref, *, cdt):
    o_ref[...] = jnp.dot(a_ref[...].astype(cdt), b_ref[...].astype(cdt),
                         preferred_element_type=jnp.float32)


def _matmul(a, b, bn=512, cdt=jnp.float32):
    m, kk = a.shape
    _, n = b.shape
    return pl.pallas_call(
        functools.partial(_mm_body, cdt=cdt),
        grid=(n // bn,),
        in_specs=[
            pl.BlockSpec((m, kk), lambda j: (0, 0)),
            pl.BlockSpec((kk, bn), lambda j: (0, j)),
        ],
        out_specs=pl.BlockSpec((m, bn), lambda j: (0, j)),
        out_shape=jax.ShapeDtypeStruct((m, n), jnp.float32),
    )(a, b)


def _indexer_body(q_ref, k_ref, wq_ref, wk_ref, qi_ref, ki_ref, *, hd):
    qsum = q_ref[:, 0:hd]
    ksum = k_ref[:, 0:hd]
    for h in range(1, H):
        qsum = qsum + q_ref[:, h * hd:(h + 1) * hd]
        ksum = ksum + k_ref[:, h * hd:(h + 1) * hd]
    qm = qsum * (1.0 / H)
    km = ksum * (1.0 / H)
    qi_ref[...] = jnp.dot(qm, wq_ref[...], preferred_element_type=jnp.float32)
    ki_ref[...] = jnp.dot(km, wk_ref[...], preferred_element_type=jnp.float32)


def _indexer(qkv, wq, wk, s, d, hd, bq=256):
    grid = (s // bq,)
    return pl.pallas_call(
        functools.partial(_indexer_body, hd=hd),
        grid=grid,
        in_specs=[
            pl.BlockSpec((bq, d), lambda i: (i, 0)),
            pl.BlockSpec((bq, d), lambda i: (i, 1)),
            pl.BlockSpec((hd, RANK), lambda i: (0, 0)),
            pl.BlockSpec((hd, RANK), lambda i: (0, 0)),
        ],
        out_specs=[
            pl.BlockSpec((bq, RANK), lambda i: (i, 0)),
            pl.BlockSpec((bq, RANK), lambda i: (i, 0)),
        ],
        out_shape=[
            jax.ShapeDtypeStruct((s, RANK), jnp.float32),
            jax.ShapeDtypeStruct((s, RANK), jnp.float32),
        ],
    )(qkv, qkv, wq, wk)


def _mask_body(qi_ref, ki_ref, mask_ref):
    scores = lax.dot_general(
        qi_ref[...], ki_ref[...], (((1,), (1,)), ((), ())),
        preferred_element_type=jnp.float32) * (1.0 / np.sqrt(RANK))
    bits = lax.bitcast_convert_type(scores, jnp.int32)
    skey = jnp.where(bits >= 0, bits, bits ^ _INT_MAXP)
    t = jnp.zeros(skey.shape[:1] + (1,), jnp.int32)
    for bit in range(31, -1, -1):
        cand = t | np.int32(np.uint32(1 << bit))
        scand = cand ^ _INT_MIN
        cnt = jnp.sum((skey >= scand).astype(jnp.int32), axis=1,
                      keepdims=True)
        t = jnp.where(cnt >= TOPK, cand, t)
    mask_ref[...] = (skey >= (t ^ _INT_MIN)).astype(jnp.int8)


def _topk_mask(qi, ki, s, bq=256):
    return pl.pallas_call(
        _mask_body,
        grid=(s // bq,),
        in_specs=[
            pl.BlockSpec((bq, RANK), lambda i: (i, 0)),
            pl.BlockSpec((s, RANK), lambda i: (0, 0)),
        ],
        out_specs=pl.BlockSpec((bq, s), lambda i: (i, 0)),
        out_shape=jax.ShapeDtypeStruct((s, s), jnp.int8),
    )(qi, ki)


def _attn_body(q_ref, k_ref, v_ref, m_ref, o_ref, *, scale, hd, hpb):
    bias = (m_ref[...].astype(jnp.float32) - 1.0) * 1e30
    for h in range(hpb):
        q = (q_ref[:, h * hd:(h + 1) * hd] * scale).astype(jnp.bfloat16)
        k = k_ref[:, h * hd:(h + 1) * hd].astype(jnp.bfloat16)
        s = lax.dot_general(q, k, (((1,), (1,)), ((), ())),
                            preferred_element_type=jnp.float32)
        s = s + bias
        mx = jnp.max(s, axis=1, keepdims=True)
        e = jnp.exp(s - mx).astype(jnp.bfloat16)
        inv = 1.0 / jnp.sum(e.astype(jnp.float32), axis=1, keepdims=True)
        o = jnp.dot(e, v_ref[:, h * hd:(h + 1) * hd].astype(jnp.bfloat16),
                    preferred_element_type=jnp.float32)
        o_ref[:, h * hd:(h + 1) * hd] = o * inv


def _attention(qkv, mask, s, hd, bq=512, hpb=8):
    scale = 1.0 / np.sqrt(hd)
    bq = min(bq, s)
    w = hpb * hd
    npair = H // hpb
    return pl.pallas_call(
        functools.partial(_attn_body, scale=scale, hd=hd, hpb=hpb),
        grid=(npair, s // bq),
        in_specs=[
            pl.BlockSpec((bq, w), lambda h, i: (i, h)),
            pl.BlockSpec((s, w), lambda h, i: (0, npair + h)),
            pl.BlockSpec((s, w), lambda h, i: (0, 2 * npair + h)),
            pl.BlockSpec((bq, s), lambda h, i: (i, 0)),
        ],
        out_specs=pl.BlockSpec((bq, w), lambda h, i: (i, h)),
        out_shape=jax.ShapeDtypeStruct((s, H * hd), jnp.float32),
    )(qkv, qkv, qkv, mask)


def kernel(x, W_qkv, W_out, Wq_idx, Wk_idx):
    b, s, d = x.shape
    hd = d // H
    x2 = x.reshape(s, d)
    qkv = _matmul(x2, W_qkv)
    qi, ki = _indexer(qkv, Wq_idx, Wk_idx, s, d, hd)
    mask = _topk_mask(qi, ki, s)
    attn = _attention(qkv, mask, s, hd)
    out = _matmul(attn, W_out)
    return out.reshape(b, s, d)
